# baseline (speedup 1.0000x reference)
import jax
import jax.numpy as jnp
from jax import lax
from jax.experimental import pallas as pl
from jax.experimental.pallas import tpu as pltpu
from jax.experimental.pallas import tpu_sc as plsc

N = 10000
E = 320000
DF = 128
ED = 16
H = 4
C = 128
HC = H * C
NR = 51
NRP = 64
DW = 144

NC = 2
NS = 16
NW = NC * NS

EPW = E // NW
EPS = E // NS
NPS = N // NS

BA = 400
BB = 80
BD = 80

def _mesh():
    return plsc.VectorSubcoreMesh(core_axis_name="c", subcore_axis_name="s",
                                  num_cores=NC, num_subcores=NS)


def _iota16():
    return lax.iota(jnp.int32, 16)


def _full16(v):
    return jnp.full((16,), v, jnp.int32)


def _tcw_body(We1, ae1, We2, ae2, Wd1, Wd2p, bd1, bd2p,
              Ve1_o, Ve2_o, Wcs_o, Wcd_o, cb_o):
    Ve1_o[...] = jnp.sum(We1[...].reshape(ED, H, C) * ae1[...][None], axis=-1)
    Ve2_o[...] = jnp.sum(We2[...].reshape(ED, H, C) * ae2[...][None], axis=-1)
    Wcs_o[...] = jnp.dot(Wd1[0:HC, :], Wd2p[...],
                         preferred_element_type=jnp.float32)
    Wcd_o[...] = jnp.dot(Wd1[HC:2 * HC, :], Wd2p[...],
                         preferred_element_type=jnp.float32)
    cb_o[...] = jnp.dot(bd1[...].reshape(1, 256), Wd2p[...],
                        preferred_element_type=jnp.float32) + bd2p[...].reshape(1, NRP)


def _tc_weights(We1, ae1, We2, ae2, Wd1, Wd2p, bd1, bd2p):
    return pl.pallas_call(
        _tcw_body,
        out_shape=(
            jax.ShapeDtypeStruct((ED, H), jnp.float32),
            jax.ShapeDtypeStruct((ED, H), jnp.float32),
            jax.ShapeDtypeStruct((HC, NRP), jnp.float32),
            jax.ShapeDtypeStruct((HC, NRP), jnp.float32),
            jax.ShapeDtypeStruct((1, NRP), jnp.float32),
        ),
    )(We1, ae1, We2, ae2, Wd1, Wd2p, bd1, bd2p)


_NB = 1000


def _write_hp(hp_o, h3):
    pat = (lax.broadcasted_iota(jnp.int32, (_NB, H, DW - C), 2) == 0)
    hp_o[...] = jnp.concatenate([h3, pat.astype(jnp.float32)], axis=-1)


def _als_ald(h3, a_s, a_d, als_o, ald_o):
    als_o[...] = jnp.sum(h3 * a_s[...][None], axis=-1)
    ald_o[...] = jnp.sum(h3 * a_d[...][None], axis=-1)


def _tc1_body(x, W, a_s, a_d, hp_o, als_o, ald_o):
    h = jnp.dot(x[...], W[...], preferred_element_type=jnp.float32)
    h3 = h.reshape(_NB, H, C)
    _write_hp(hp_o, h3)
    _als_ald(h3, a_s, a_d, als_o, ald_o)


def _tc_dense1(x, W1, as1, ad1):
    grid = (N // _NB,)
    return pl.pallas_call(
        _tc1_body,
        grid=grid,
        in_specs=[
            pl.BlockSpec((_NB, DF), lambda i: (i, 0)),
            pl.BlockSpec((DF, HC), lambda i: (0, 0)),
            pl.BlockSpec((H, C), lambda i: (0, 0)),
            pl.BlockSpec((H, C), lambda i: (0, 0)),
        ],
        out_specs=(
            pl.BlockSpec((_NB, H, DW), lambda i: (i, 0, 0)),
            pl.BlockSpec((_NB, H), lambda i: (i, 0)),
            pl.BlockSpec((_NB, H), lambda i: (i, 0)),
        ),
        out_shape=(
            jax.ShapeDtypeStruct((N, H, DW), jnp.float32),
            jax.ShapeDtypeStruct((N, H), jnp.float32),
            jax.ShapeDtypeStruct((N, H), jnp.float32),
        ),
    )(x, W1, as1, ad1)


def _z_from_acc(acc, b):
    num = acc[:, :, 0:C]
    den = acc[:, :, C:C + 1]
    z = num / (den + 1e-16) + b[...].reshape(1, H, C)
    return jnp.where(z > 0, z, jnp.exp(z) - 1.0)


def _tc2_body(acc, b, W, a_s, a_d, hp_o, als_o, ald_o):
    z3 = _z_from_acc(acc[...], b)
    z = z3.reshape(_NB, HC)
    h = jnp.dot(z, W[...], preferred_element_type=jnp.float32)
    h3 = h.reshape(_NB, H, C)
    _write_hp(hp_o, h3)
    _als_ald(h3, a_s, a_d, als_o, ald_o)


def _tc_dense2(acc1, b1, W2, as2, ad2):
    grid = (N // _NB,)
    return pl.pallas_call(
        _tc2_body,
        grid=grid,
        in_specs=[
            pl.BlockSpec((_NB, H, DW), lambda i: (i, 0, 0)),
            pl.BlockSpec((HC,), lambda i: (0,)),
            pl.BlockSpec((HC, HC), lambda i: (0, 0)),
            pl.BlockSpec((H, C), lambda i: (0, 0)),
            pl.BlockSpec((H, C), lambda i: (0, 0)),
        ],
        out_specs=(
            pl.BlockSpec((_NB, H, DW), lambda i: (i, 0, 0)),
            pl.BlockSpec((_NB, H), lambda i: (i, 0)),
            pl.BlockSpec((_NB, H), lambda i: (i, 0)),
        ),
        out_shape=(
            jax.ShapeDtypeStruct((N, H, DW), jnp.float32),
            jax.ShapeDtypeStruct((N, H), jnp.float32),
            jax.ShapeDtypeStruct((N, H), jnp.float32),
        ),
    )(acc1, b1, W2, as2, ad2)


def _tc3_body(acc, b, Wcs, Wcd, cb, Qs_o, Qd_o):
    z3 = _z_from_acc(acc[...], b)
    z = z3.reshape(_NB, HC)
    Qs_o[...] = jnp.dot(z, Wcs[...], preferred_element_type=jnp.float32) + cb[...]
    Qd_o[...] = jnp.dot(z, Wcd[...], preferred_element_type=jnp.float32)


def _tc_dec(acc2, b2, Wcs, Wcd, cb):
    grid = (N // _NB,)
    return pl.pallas_call(
        _tc3_body,
        grid=grid,
        in_specs=[
            pl.BlockSpec((_NB, H, DW), lambda i: (i, 0, 0)),
            pl.BlockSpec((HC,), lambda i: (0,)),
            pl.BlockSpec((HC, NRP), lambda i: (0, 0)),
            pl.BlockSpec((HC, NRP), lambda i: (0, 0)),
            pl.BlockSpec((1, NRP), lambda i: (0, 0)),
        ],
        out_specs=(
            pl.BlockSpec((_NB, NRP), lambda i: (i, 0)),
            pl.BlockSpec((_NB, NRP), lambda i: (i, 0)),
        ),
        out_shape=(
            jax.ShapeDtypeStruct((N, NRP), jnp.float32),
            jax.ShapeDtypeStruct((N, NRP), jnp.float32),
        ),
    )(acc2, b2, Wcs, Wcd, cb)


_EB = 8000


def _tce_body(ea, Ve1, Ve2, ale1_o, ale2_o):
    e = ea[...]
    ale1_o[...] = jnp.dot(e, Ve1[...], preferred_element_type=jnp.float32)
    ale2_o[...] = jnp.dot(e, Ve2[...], preferred_element_type=jnp.float32)


def _tc_ale(ea, Ve1, Ve2):
    grid = (E // _EB,)
    return pl.pallas_call(
        _tce_body,
        grid=grid,
        in_specs=[
            pl.BlockSpec((_EB, ED), lambda i: (i, 0)),
            pl.BlockSpec((ED, H), lambda i: (0, 0)),
            pl.BlockSpec((ED, H), lambda i: (0, 0)),
        ],
        out_specs=(
            pl.BlockSpec((_EB, H), lambda i: (i, 0)),
            pl.BlockSpec((_EB, H), lambda i: (i, 0)),
        ),
        out_shape=(
            jax.ShapeDtypeStruct((E, H), jnp.float32),
            jax.ShapeDtypeStruct((E, H), jnp.float32),
        ),
    )(ea, Ve1, Ve2)


def _sc_logits_body(esrc, edst, als, ald, ale, g_o,
                    als_v, ald_v, src_v, dst_v, ale_v, g_v):
    wid = lax.axis_index("s") * NC + lax.axis_index("c")
    base = wid * EPW
    pltpu.sync_copy(als, als_v)
    pltpu.sync_copy(ald, ald_v)
    it = _iota16()

    def block(b, _):
        e0 = base + b * BA
        pltpu.sync_copy(esrc.at[pl.ds(e0, BA)], src_v)
        pltpu.sync_copy(edst.at[pl.ds(e0, BA)], dst_v)
        pltpu.sync_copy(ale.at[pl.ds(e0 * H, BA * H)], ale_v)

        def group(i, _):
            off16 = i * 16 + it
            s16 = src_v[pl.ds(i * 16, 16)]
            d16 = dst_v[pl.ds(i * 16, 16)]
            for h in range(H):
                t = (plsc.load_gather(als_v, [s16 * H + h])
                     + plsc.load_gather(ald_v, [d16 * H + h])
                     + plsc.load_gather(ale_v, [off16 * H + h]))
                t = jnp.where(t >= 0, t, t * 0.2)
                plsc.store_scatter(g_v, [off16 * H + h], jnp.exp(t))
            return 0

        lax.fori_loop(0, BA // 16, group, 0)
        pltpu.sync_copy(g_v, g_o.at[pl.ds(e0 * H, BA * H)])
        return 0

    lax.fori_loop(0, EPW // BA, block, 0)


def _sc_logits(esrc, edst, als, ald, ale):
    kern = pl.kernel(
        _sc_logits_body,
        out_type=jax.ShapeDtypeStruct((E * H,), jnp.float32),
        mesh=_mesh(),
        compiler_params=pltpu.CompilerParams(use_tc_tiling_on_sc=False, needs_layout_passes=False),
        scratch_types=[
            pltpu.VMEM((N * H,), jnp.float32),
            pltpu.VMEM((N * H,), jnp.float32),
            pltpu.VMEM((BA,), jnp.int32),
            pltpu.VMEM((BA,), jnp.int32),
            pltpu.VMEM((BA * H,), jnp.float32),
            pltpu.VMEM((BA * H,), jnp.float32),
        ],
    )
    return kern(esrc, edst, als, ald, ale)


NSUB = 2
SB = NSUB * BB


def _sc_agg_body(esrc, edst2, hp, g, acc_o,
                 acc_s, rows_v, sbuf_v, src_v, dst_v, idx_v, g_v, gsem):
    c = lax.axis_index("c")
    s = lax.axis_index("s")

    for hh in range(2):
        h = c * 2 + hh
        def zrow(r, _):
            for j in range(DW // 16):
                sbuf_v[r, pl.ds(j * 16, 16)] = jnp.zeros((16,), jnp.float32)
            return 0
        lax.fori_loop(0, BB, zrow, 0)
        for k in range(7):
            pltpu.sync_copy(sbuf_v, acc_s.at[pl.ds(s * NPS + k * BB, BB), :])
        pltpu.sync_copy(sbuf_v.at[pl.ds(0, NPS - 7 * BB), :],
                        acc_s.at[pl.ds(s * NPS + 7 * BB, NPS - 7 * BB), :])
        plsc.subcore_barrier()

        def superblock(b, _):
            e0 = s * EPS + b * SB
            pltpu.sync_copy(esrc.at[pl.ds(e0, SB)], src_v)
            pltpu.sync_copy(edst2.at[pl.ds(e0 // BB, NSUB), :], dst_v)
            pltpu.sync_copy(g.at[pl.ds(e0 * H, SB * H)], g_v)
            for i in range(SB // 16):
                s16 = src_v[pl.ds(i * 16, 16)]
                idx_v[pl.ds(i * 16, 16)] = s16 * H + h

            def gfire(sub, _):
                pltpu.async_copy(
                    hp.at[idx_v.at[pl.ds(sub * BB, BB)]],
                    rows_v.at[pl.ds(sub * BB, BB), :], gsem)
                return 0
            lax.fori_loop(0, NSUB, gfire, 0)

            def gdrain(sub, _):
                pltpu.make_async_copy(
                    hp.at[idx_v.at[pl.ds(sub * BB, BB)]],
                    rows_v.at[pl.ds(sub * BB, BB), :], gsem).wait()
                return 0
            lax.fori_loop(0, NSUB, gdrain, 0)

            def scat(sub, _):
                def row(p, _):
                    r0 = p * 2
                    ga = plsc.load_gather(g_v, [_full16((sub * BB + r0) * H + h)])
                    gb = plsc.load_gather(g_v, [_full16((sub * BB + r0 + 1) * H + h)])
                    for j in range(DW // 16):
                        sl = pl.ds(j * 16, 16)
                        sbuf_v[r0, sl] = rows_v[sub * BB + r0, sl] * ga
                        sbuf_v[r0 + 1, sl] = rows_v[sub * BB + r0 + 1, sl] * gb
                    return 0
                lax.fori_loop(0, BB // 2, row, 0)
                pltpu.sync_copy(sbuf_v, acc_s.at[dst_v.at[sub]], add=True)
                return 0
            lax.fori_loop(0, NSUB, scat, 0)
            return 0

        lax.fori_loop(0, EPS // SB, superblock, 0)
        plsc.subcore_barrier()
        pltpu.sync_copy(acc_s.at[pl.ds(s * NPS, NPS), :],
                        acc_o.at[pl.ds(s * NPS, NPS), h, :])
        plsc.subcore_barrier()


def _sc_agg(esrc, edst, hp2d, g):
    kern = pl.kernel(
        _sc_agg_body,
        out_type=jax.ShapeDtypeStruct((N, H, DW), jnp.float32),
        mesh=_mesh(),
        compiler_params=pltpu.CompilerParams(use_tc_tiling_on_sc=False, needs_layout_passes=False),
        scratch_types=[
            pltpu.VMEM_SHARED((N, DW), jnp.float32),
            pltpu.VMEM((SB, DW), jnp.float32),
            pltpu.VMEM((BB, DW), jnp.float32),
            pltpu.VMEM((SB,), jnp.int32),
            pltpu.VMEM((NSUB, BB), jnp.int32),
            pltpu.VMEM((SB,), jnp.int32),
            pltpu.VMEM((SB * H,), jnp.float32),
            pltpu.SemaphoreType.DMA,
        ],
    )
    return kern(esrc, edst.reshape(E // BB, BB), hp2d, g)


def _sc_decode_body(esrc, edst, Qs, Qd, out_o,
                    src_v, dst_v, qs_v, qd_v, obuf, sem1, sem2):
    wid = lax.axis_index("s") * NC + lax.axis_index("c")
    base = wid * EPW

    def block(b, _):
        e0 = base + b * BD
        pltpu.sync_copy(esrc.at[pl.ds(e0, BD)], src_v)
        pltpu.sync_copy(edst.at[pl.ds(e0, BD)], dst_v)
        cp1 = pltpu.async_copy(Qs.at[src_v], qs_v, sem1)
        cp2 = pltpu.async_copy(Qd.at[dst_v], qd_v, sem2)
        cp1.wait()
        cp2.wait()

        it = _iota16()

        def row(r, _):
            for j in range(4):
                sl = pl.ds(j * 16, 16)
                v = qs_v[r, sl] + qd_v[r, sl]
                addr = r * NR + j * 16 + it
                if j < 3:
                    plsc.store_scatter(obuf, [addr], v)
                else:
                    plsc.store_scatter(obuf, [addr], v, mask=it < 3)
            return 0
        lax.fori_loop(0, BD, row, 0)
        pltpu.sync_copy(obuf, out_o.at[pl.ds(e0 * NR, BD * NR)])
        return 0

    lax.fori_loop(0, EPW // BD, block, 0)


def _sc_decode(esrc, edst, Qs, Qd):
    kern = pl.kernel(
        _sc_decode_body,
        out_type=jax.ShapeDtypeStruct((E * NR,), jnp.float32),
        mesh=_mesh(),
        compiler_params=pltpu.CompilerParams(use_tc_tiling_on_sc=False, needs_layout_passes=False),
        scratch_types=[
            pltpu.VMEM((BD,), jnp.int32),
            pltpu.VMEM((BD,), jnp.int32),
            pltpu.VMEM((BD, NRP), jnp.float32),
            pltpu.VMEM((BD, NRP), jnp.float32),
            pltpu.VMEM((BD * NR,), jnp.float32),
            pltpu.SemaphoreType.DMA,
            pltpu.SemaphoreType.DMA,
        ],
    )
    return kern(esrc, edst, Qs, Qd)


def kernel(x, edge_index, edge_features, W1, b1, as1, ad1, We1, ae1,
           W2, b2, as2, ad2, We2, ae2, Wd1, bd1, Wd2, bd2):
    esrc = edge_index[0]
    edst = edge_index[1]
    Wd2p = jnp.pad(Wd2, ((0, 0), (0, NRP - NR)))
    bd2p = jnp.pad(bd2, (0, NRP - NR))

    Ve1, Ve2, Wcs, Wcd, cb = _tc_weights(We1, ae1, We2, ae2, Wd1, Wd2p, bd1, bd2p)
    hp1, als1, ald1 = _tc_dense1(x, W1, as1, ad1)
    ale1, ale2 = _tc_ale(edge_features, Ve1, Ve2)
    g1 = _sc_logits(esrc, edst, als1.reshape(-1), ald1.reshape(-1),
                    ale1.reshape(-1))
    acc1 = _sc_agg(esrc, edst, hp1.reshape(N * H, DW), g1)

    hp2, als2, ald2 = _tc_dense2(acc1, b1, W2, as2, ad2)
    g2 = _sc_logits(esrc, edst, als2.reshape(-1), ald2.reshape(-1),
                    ale2.reshape(-1))
    acc2 = _sc_agg(esrc, edst, hp2.reshape(N * H, DW), g2)

    Qs, Qd = _tc_dec(acc2, b2, Wcs, Wcd, cb)
    return _sc_decode(esrc, edst, Qs, Qd).reshape(E, NR)


# speedup vs baseline: 1.6526x; 1.6526x over previous
import jax
import jax.numpy as jnp
from jax import lax
from jax.experimental import pallas as pl
from jax.experimental.pallas import tpu as pltpu
from jax.experimental.pallas import tpu_sc as plsc

N = 10000
E = 320000
DF = 128
ED = 16
H = 4
C = 128
HC = H * C
NR = 51
NRP = 64
DW = 144

NC = 2
NS = 16
NW = NC * NS

EPW = E // NW
EPS = E // NS
NPS = N // NS

BA = 400
BB = 80
BD = 80

def _mesh():
    return plsc.VectorSubcoreMesh(core_axis_name="c", subcore_axis_name="s",
                                  num_cores=NC, num_subcores=NS)


def _iota16():
    return lax.iota(jnp.int32, 16)


def _full16(v):
    return jnp.full((16,), v, jnp.int32)


def _tcw_body(We1, ae1, We2, ae2, Wd1, Wd2p, bd1, bd2p,
              Ve1_o, Ve2_o, Wcs_o, Wcd_o, cb_o):
    Ve1_o[...] = jnp.sum(We1[...].reshape(ED, H, C) * ae1[...][None], axis=-1)
    Ve2_o[...] = jnp.sum(We2[...].reshape(ED, H, C) * ae2[...][None], axis=-1)
    Wcs_o[...] = jnp.dot(Wd1[0:HC, :], Wd2p[...],
                         preferred_element_type=jnp.float32)
    Wcd_o[...] = jnp.dot(Wd1[HC:2 * HC, :], Wd2p[...],
                         preferred_element_type=jnp.float32)
    cb_o[...] = jnp.dot(bd1[...].reshape(1, 256), Wd2p[...],
                        preferred_element_type=jnp.float32) + bd2p[...].reshape(1, NRP)


def _tc_weights(We1, ae1, We2, ae2, Wd1, Wd2p, bd1, bd2p):
    return pl.pallas_call(
        _tcw_body,
        out_shape=(
            jax.ShapeDtypeStruct((ED, H), jnp.float32),
            jax.ShapeDtypeStruct((ED, H), jnp.float32),
            jax.ShapeDtypeStruct((HC, NRP), jnp.float32),
            jax.ShapeDtypeStruct((HC, NRP), jnp.float32),
            jax.ShapeDtypeStruct((1, NRP), jnp.float32),
        ),
    )(We1, ae1, We2, ae2, Wd1, Wd2p, bd1, bd2p)


_NB = 1000


def _write_hp(hp_o, h3):
    pat = (lax.broadcasted_iota(jnp.int32, (_NB, H, DW - C), 2) == 0)
    hp_o[...] = jnp.concatenate([h3, pat.astype(jnp.float32)], axis=-1)


def _als_ald(h3, a_s, a_d, als_o, ald_o):
    als_o[...] = jnp.sum(h3 * a_s[...][None], axis=-1)
    ald_o[...] = jnp.sum(h3 * a_d[...][None], axis=-1)


def _tc1_body(x, W, a_s, a_d, hp_o, als_o, ald_o):
    h = jnp.dot(x[...], W[...], preferred_element_type=jnp.float32)
    h3 = h.reshape(_NB, H, C)
    _write_hp(hp_o, h3)
    _als_ald(h3, a_s, a_d, als_o, ald_o)


def _tc_dense1(x, W1, as1, ad1):
    grid = (N // _NB,)
    return pl.pallas_call(
        _tc1_body,
        grid=grid,
        in_specs=[
            pl.BlockSpec((_NB, DF), lambda i: (i, 0)),
            pl.BlockSpec((DF, HC), lambda i: (0, 0)),
            pl.BlockSpec((H, C), lambda i: (0, 0)),
            pl.BlockSpec((H, C), lambda i: (0, 0)),
        ],
        out_specs=(
            pl.BlockSpec((_NB, H, DW), lambda i: (i, 0, 0)),
            pl.BlockSpec((_NB, H), lambda i: (i, 0)),
            pl.BlockSpec((_NB, H), lambda i: (i, 0)),
        ),
        out_shape=(
            jax.ShapeDtypeStruct((N, H, DW), jnp.float32),
            jax.ShapeDtypeStruct((N, H), jnp.float32),
            jax.ShapeDtypeStruct((N, H), jnp.float32),
        ),
    )(x, W1, as1, ad1)


def _z_from_acc(acc, b):
    num = acc[:, :, 0:C]
    den = acc[:, :, C:C + 1]
    z = num / (den + 1e-16) + b[...].reshape(1, H, C)
    return jnp.where(z > 0, z, jnp.exp(z) - 1.0)


def _tc2_body(acc, b, W, a_s, a_d, hp_o, als_o, ald_o):
    z3 = _z_from_acc(acc[...], b)
    z = z3.reshape(_NB, HC)
    h = jnp.dot(z, W[...], preferred_element_type=jnp.float32)
    h3 = h.reshape(_NB, H, C)
    _write_hp(hp_o, h3)
    _als_ald(h3, a_s, a_d, als_o, ald_o)


def _tc_dense2(acc1, b1, W2, as2, ad2):
    grid = (N // _NB,)
    return pl.pallas_call(
        _tc2_body,
        grid=grid,
        in_specs=[
            pl.BlockSpec((_NB, H, DW), lambda i: (i, 0, 0)),
            pl.BlockSpec((HC,), lambda i: (0,)),
            pl.BlockSpec((HC, HC), lambda i: (0, 0)),
            pl.BlockSpec((H, C), lambda i: (0, 0)),
            pl.BlockSpec((H, C), lambda i: (0, 0)),
        ],
        out_specs=(
            pl.BlockSpec((_NB, H, DW), lambda i: (i, 0, 0)),
            pl.BlockSpec((_NB, H), lambda i: (i, 0)),
            pl.BlockSpec((_NB, H), lambda i: (i, 0)),
        ),
        out_shape=(
            jax.ShapeDtypeStruct((N, H, DW), jnp.float32),
            jax.ShapeDtypeStruct((N, H), jnp.float32),
            jax.ShapeDtypeStruct((N, H), jnp.float32),
        ),
    )(acc1, b1, W2, as2, ad2)


def _tc3_body(acc, b, Wcs, Wcd, cb, Qs_o, Qd_o):
    z3 = _z_from_acc(acc[...], b)
    z = z3.reshape(_NB, HC)
    Qs_o[...] = jnp.dot(z, Wcs[...], preferred_element_type=jnp.float32) + cb[...]
    Qd_o[...] = jnp.dot(z, Wcd[...], preferred_element_type=jnp.float32)


def _tc_dec(acc2, b2, Wcs, Wcd, cb):
    grid = (N // _NB,)
    return pl.pallas_call(
        _tc3_body,
        grid=grid,
        in_specs=[
            pl.BlockSpec((_NB, H, DW), lambda i: (i, 0, 0)),
            pl.BlockSpec((HC,), lambda i: (0,)),
            pl.BlockSpec((HC, NRP), lambda i: (0, 0)),
            pl.BlockSpec((HC, NRP), lambda i: (0, 0)),
            pl.BlockSpec((1, NRP), lambda i: (0, 0)),
        ],
        out_specs=(
            pl.BlockSpec((_NB, NRP), lambda i: (i, 0)),
            pl.BlockSpec((_NB, NRP), lambda i: (i, 0)),
        ),
        out_shape=(
            jax.ShapeDtypeStruct((N, NRP), jnp.float32),
            jax.ShapeDtypeStruct((N, NRP), jnp.float32),
        ),
    )(acc2, b2, Wcs, Wcd, cb)


_EB = 8000


def _tce_body(ea, Ve1, Ve2, ale1_o, ale2_o):
    e = ea[...]
    ale1_o[...] = jnp.dot(e, Ve1[...], preferred_element_type=jnp.float32)
    ale2_o[...] = jnp.dot(e, Ve2[...], preferred_element_type=jnp.float32)


def _tc_ale(ea, Ve1, Ve2):
    grid = (E // _EB,)
    return pl.pallas_call(
        _tce_body,
        grid=grid,
        in_specs=[
            pl.BlockSpec((_EB, ED), lambda i: (i, 0)),
            pl.BlockSpec((ED, H), lambda i: (0, 0)),
            pl.BlockSpec((ED, H), lambda i: (0, 0)),
        ],
        out_specs=(
            pl.BlockSpec((_EB, H), lambda i: (i, 0)),
            pl.BlockSpec((_EB, H), lambda i: (i, 0)),
        ),
        out_shape=(
            jax.ShapeDtypeStruct((E, H), jnp.float32),
            jax.ShapeDtypeStruct((E, H), jnp.float32),
        ),
    )(ea, Ve1, Ve2)


def _sc_logits_body(esrc, edst, als, ald, ale, g_o,
                    als_v, ald_v, src_v, dst_v, ale_v, g_v):
    wid = lax.axis_index("s") * NC + lax.axis_index("c")
    base = wid * EPW
    pltpu.sync_copy(als, als_v)
    pltpu.sync_copy(ald, ald_v)
    it = _iota16()

    def block(b, _):
        e0 = base + b * BA
        pltpu.sync_copy(esrc.at[pl.ds(e0, BA)], src_v)
        pltpu.sync_copy(edst.at[pl.ds(e0, BA)], dst_v)
        pltpu.sync_copy(ale.at[pl.ds(e0 * H, BA * H)], ale_v)

        def group(i, _):
            off16 = i * 16 + it
            s16 = src_v[pl.ds(i * 16, 16)]
            d16 = dst_v[pl.ds(i * 16, 16)]
            for h in range(H):
                t = (plsc.load_gather(als_v, [s16 * H + h])
                     + plsc.load_gather(ald_v, [d16 * H + h])
                     + plsc.load_gather(ale_v, [off16 * H + h]))
                t = jnp.where(t >= 0, t, t * 0.2)
                plsc.store_scatter(g_v, [off16 * H + h], jnp.exp(t))
            return 0

        lax.fori_loop(0, BA // 16, group, 0)
        pltpu.sync_copy(g_v, g_o.at[pl.ds(e0 * H, BA * H)])
        return 0

    lax.fori_loop(0, EPW // BA, block, 0)


def _sc_logits(esrc, edst, als, ald, ale):
    kern = pl.kernel(
        _sc_logits_body,
        out_type=jax.ShapeDtypeStruct((E * H,), jnp.float32),
        mesh=_mesh(),
        compiler_params=pltpu.CompilerParams(use_tc_tiling_on_sc=False, needs_layout_passes=False),
        scratch_types=[
            pltpu.VMEM((N * H,), jnp.float32),
            pltpu.VMEM((N * H,), jnp.float32),
            pltpu.VMEM((BA,), jnp.int32),
            pltpu.VMEM((BA,), jnp.int32),
            pltpu.VMEM((BA * H,), jnp.float32),
            pltpu.VMEM((BA * H,), jnp.float32),
        ],
    )
    return kern(esrc, edst, als, ald, ale)


NSUB = 2
SB = NSUB * BB


def _sc_agg_body(esrc, edst2, hp, g, acc_o,
                 acc_s, rows_v, sbuf_v, src_v, dst_v, idx_v, g_v, gsem):
    c = lax.axis_index("c")
    s = lax.axis_index("s")

    for hh in range(2):
        h = c * 2 + hh
        def zrow(r, _):
            for j in range(DW // 16):
                sbuf_v[r, pl.ds(j * 16, 16)] = jnp.zeros((16,), jnp.float32)
            return 0
        lax.fori_loop(0, BB, zrow, 0)
        for k in range(7):
            pltpu.sync_copy(sbuf_v, acc_s.at[pl.ds(s * NPS + k * BB, BB), :])
        pltpu.sync_copy(sbuf_v.at[pl.ds(0, NPS - 7 * BB), :],
                        acc_s.at[pl.ds(s * NPS + 7 * BB, NPS - 7 * BB), :])
        plsc.subcore_barrier()

        def superblock(b, _):
            e0 = s * EPS + b * SB
            pltpu.sync_copy(esrc.at[pl.ds(e0, SB)], src_v)
            pltpu.sync_copy(edst2.at[pl.ds(e0 // BB, NSUB), :], dst_v)
            pltpu.sync_copy(g.at[pl.ds(e0 * H, SB * H)], g_v)
            for i in range(SB // 16):
                s16 = src_v[pl.ds(i * 16, 16)]
                idx_v[pl.ds(i * 16, 16)] = s16 * H + h

            def gfire(sub, _):
                pltpu.async_copy(
                    hp.at[idx_v.at[pl.ds(sub * BB, BB)]],
                    rows_v.at[pl.ds(sub * BB, BB), :], gsem)
                return 0
            lax.fori_loop(0, NSUB, gfire, 0)

            def gdrain(sub, _):
                pltpu.make_async_copy(
                    hp.at[idx_v.at[pl.ds(sub * BB, BB)]],
                    rows_v.at[pl.ds(sub * BB, BB), :], gsem).wait()
                return 0
            lax.fori_loop(0, NSUB, gdrain, 0)

            def scat(sub, _):
                def row(r, _):
                    gs = plsc.load_gather(g_v, [_full16((sub * BB + r) * H + h)])
                    for j in range(DW // 16):
                        sl = pl.ds(j * 16, 16)
                        sbuf_v[r, sl] = rows_v[sub * BB + r, sl] * gs
                    return 0
                lax.fori_loop(0, BB, row, 0)
                pltpu.sync_copy(sbuf_v, acc_s.at[dst_v.at[sub]], add=True)
                return 0
            lax.fori_loop(0, NSUB, scat, 0)
            return 0

        lax.fori_loop(0, EPS // SB, superblock, 0)
        plsc.subcore_barrier()
        pltpu.sync_copy(acc_s.at[pl.ds(s * NPS, NPS), :],
                        acc_o.at[pl.ds(s * NPS, NPS), h, :])
        plsc.subcore_barrier()


def _sc_agg(esrc, edst, hp2d, g):
    kern = pl.kernel(
        _sc_agg_body,
        out_type=jax.ShapeDtypeStruct((N, H, DW), jnp.float32),
        mesh=_mesh(),
        compiler_params=pltpu.CompilerParams(use_tc_tiling_on_sc=False, needs_layout_passes=False),
        scratch_types=[
            pltpu.VMEM_SHARED((N, DW), jnp.float32),
            pltpu.VMEM((SB, DW), jnp.float32),
            pltpu.VMEM((BB, DW), jnp.float32),
            pltpu.VMEM((SB,), jnp.int32),
            pltpu.VMEM((NSUB, BB), jnp.int32),
            pltpu.VMEM((SB,), jnp.int32),
            pltpu.VMEM((SB * H,), jnp.float32),
            pltpu.SemaphoreType.DMA,
        ],
    )
    return kern(esrc, edst.reshape(E // BB, BB), hp2d, g)


def _sc_decode_body(esrc, edst, Qs, Qd, out_o,
                    src_v, dst_v, qs_v, qd_v, obuf, sem1, sem2):
    wid = lax.axis_index("s") * NC + lax.axis_index("c")
    base = wid * EPW

    def block(b, _):
        e0 = base + b * BD
        pltpu.sync_copy(esrc.at[pl.ds(e0, BD)], src_v)
        pltpu.sync_copy(edst.at[pl.ds(e0, BD)], dst_v)
        cp1 = pltpu.async_copy(Qs.at[src_v], qs_v, sem1)
        cp2 = pltpu.async_copy(Qd.at[dst_v], qd_v, sem2)
        cp1.wait()
        cp2.wait()

        it = _iota16()

        def row(r, _):
            for j in range(4):
                sl = pl.ds(j * 16, 16)
                v = qs_v[r, sl] + qd_v[r, sl]
                addr = r * NR + j * 16 + it
                if j < 3:
                    plsc.store_scatter(obuf, [addr], v)
                else:
                    plsc.store_scatter(obuf, [addr], v, mask=it < 3)
            return 0
        lax.fori_loop(0, BD, row, 0)
        pltpu.sync_copy(obuf, out_o.at[pl.ds(e0 * NR, BD * NR)])
        return 0

    lax.fori_loop(0, EPW // BD, block, 0)


def _sc_decode(esrc, edst, Qs, Qd):
    kern = pl.kernel(
        _sc_decode_body,
        out_type=jax.ShapeDtypeStruct((E * NR,), jnp.float32),
        mesh=_mesh(),
        compiler_params=pltpu.CompilerParams(use_tc_tiling_on_sc=False, needs_layout_passes=False),
        scratch_types=[
            pltpu.VMEM((BD,), jnp.int32),
            pltpu.VMEM((BD,), jnp.int32),
            pltpu.VMEM((BD, NRP), jnp.float32),
            pltpu.VMEM((BD, NRP), jnp.float32),
            pltpu.VMEM((BD * NR,), jnp.float32),
            pltpu.SemaphoreType.DMA,
            pltpu.SemaphoreType.DMA,
        ],
    )
    return kern(esrc, edst, Qs, Qd)


def kernel(x, edge_index, edge_features, W1, b1, as1, ad1, We1, ae1,
           W2, b2, as2, ad2, We2, ae2, Wd1, bd1, Wd2, bd2):
    esrc = edge_index[0]
    edst = edge_index[1]
    Wd2p = jnp.pad(Wd2, ((0, 0), (0, NRP - NR)))
    bd2p = jnp.pad(bd2, (0, NRP - NR))

    Ve1, Ve2, Wcs, Wcd, cb = _tc_weights(We1, ae1, We2, ae2, Wd1, Wd2p, bd1, bd2p)
    hp1, als1, ald1 = _tc_dense1(x, W1, as1, ad1)
    ale1, ale2 = _tc_ale(edge_features, Ve1, Ve2)
    g1 = _sc_logits(esrc, edst, als1.reshape(-1), ald1.reshape(-1),
                    ale1.reshape(-1))
    acc1 = _sc_agg(esrc, edst, hp1.reshape(N * H, DW), g1)

    hp2, als2, ald2 = _tc_dense2(acc1, b1, W2, as2, ad2)
    g2 = _sc_logits(esrc, edst, als2.reshape(-1), ald2.reshape(-1),
                    ale2.reshape(-1))
    acc2 = _sc_agg(esrc, edst, hp2.reshape(N * H, DW), g2)

    Qs, Qd = _tc_dec(acc2, b2, Wcs, Wcd, cb)
    return _sc_decode(esrc, edst, Qs, Qd).reshape(E, NR)
